# baseline (speedup 1.0000x reference)
import jax
import jax.numpy as jnp
from jax.experimental import pallas as pl
from jax.experimental.pallas import tpu as pltpu

_B, _C, _H, _W = 16, 64, 32, 32
_P = _H * _W
_K = 1024
_N = _B * _P
_COMMIT = 0.25


def _vq_body(z_ref, emb_ref, embt_ref, out_ref, idx_ref, loss_ref, perp_ref):
    emb = emb_ref[...]
    embt = embt_ref[...]
    e2 = jnp.sum(emb * emb, axis=1, keepdims=True)
    n2e = -2.0 * emb
    iota_f = jax.lax.broadcasted_iota(
        jnp.int32, (_K, _P), 0).astype(jnp.float32)

    def _one(zb):
        m2 = jax.lax.dot_general(n2e, zb, (((1,), (0,)), ((), ())),
                                 preferred_element_type=jnp.float32)
        d = m2 + e2

        minv = jnp.min(d, axis=0, keepdims=True)
        idx_f = jnp.min(jnp.where(d <= minv, iota_f, jnp.float32(_K)),
                        axis=0)
        onehot = (iota_f == idx_f[None, :]).astype(jnp.float32)
        q = jnp.dot(embt, onehot, preferred_element_type=jnp.float32)
        z2 = jnp.sum(zb * zb, axis=0)
        ploss = jnp.sum(z2 + minv[0])
        pcnt = jnp.sum(onehot, axis=1, keepdims=True)
        return idx_f.astype(jnp.int32), q, ploss, pcnt

    total_loss = jnp.float32(0)
    total_cnt = jnp.zeros((_K, 1), jnp.float32)
    for u in range(_B):
        idx_u, q_u, ploss_u, pcnt_u = _one(z_ref[u])
        idx_ref[u, 0, :] = idx_u
        out_ref[u] = q_u
        total_loss = total_loss + ploss_u
        total_cnt = total_cnt + pcnt_u

    loss_ref[0, 0] = (_COMMIT / (_N * _C)) * total_loss
    avg = total_cnt / _N
    perp_ref[0, 0] = jnp.exp(-jnp.sum(avg * jnp.log(avg + 1e-10)))


def kernel(z, embedding):
    zv = z.reshape(_B, _C, _P)
    out_q, idx, loss, perp = pl.pallas_call(
        _vq_body,
        in_specs=[
            pl.BlockSpec((_B, _C, _P), lambda: (0, 0, 0)),
            pl.BlockSpec((_K, _C), lambda: (0, 0)),
            pl.BlockSpec((_C, _K), lambda: (0, 0)),
        ],
        out_specs=[
            pl.BlockSpec((_B, _C, _P), lambda: (0, 0, 0)),
            pl.BlockSpec((_B, 1, _P), lambda: (0, 0, 0)),
            pl.BlockSpec(memory_space=pltpu.SMEM),
            pl.BlockSpec(memory_space=pltpu.SMEM),
        ],
        out_shape=[
            jax.ShapeDtypeStruct((_B, _C, _P), jnp.float32),
            jax.ShapeDtypeStruct((_B, 1, _P), jnp.int32),
            jax.ShapeDtypeStruct((1, 1), jnp.float32),
            jax.ShapeDtypeStruct((1, 1), jnp.float32),
        ],
    )(zv, embedding, embedding.T)

    return (out_q.reshape(_B, _C, _H, _W),
            loss[0, 0],
            idx.reshape(_B, _H, _W),
            perp[0, 0])


# speedup vs baseline: 1.0320x; 1.0320x over previous
import jax
import jax.numpy as jnp
from jax.experimental import pallas as pl
from jax.experimental.pallas import tpu as pltpu

_B, _C, _H, _W = 16, 64, 32, 32
_P = _H * _W
_K = 1024
_N = _B * _P
_U = 4
_COMMIT = 0.25


def _vq_body(z_ref, emb_ref, embt_ref,
             out_ref, idx_ref, loss_ref, perp_ref,
             loss_acc, cnt_acc):
    b = pl.program_id(0)

    emb = emb_ref[...]
    embt = embt_ref[...]
    e2 = jnp.sum(emb * emb, axis=1, keepdims=True)
    n2e = -2.0 * emb
    iota_f = jax.lax.broadcasted_iota(
        jnp.int32, (_K, _P), 0).astype(jnp.float32)

    def _one(zb):
        m2 = jax.lax.dot_general(n2e, zb, (((1,), (0,)), ((), ())),
                                 preferred_element_type=jnp.float32)
        d = m2 + e2

        minv = jnp.min(d, axis=0, keepdims=True)
        idx_f = jnp.min(jnp.where(d <= minv, iota_f, jnp.float32(_K)),
                        axis=0)
        onehot = (iota_f == idx_f[None, :]).astype(jnp.float32)
        q = jnp.dot(embt, onehot, preferred_element_type=jnp.float32)
        z2 = jnp.sum(zb * zb, axis=0)
        ploss = jnp.sum(z2 + minv[0])
        pcnt = jnp.sum(onehot, axis=1, keepdims=True)
        return idx_f.astype(jnp.int32), q, ploss, pcnt

    part_loss = jnp.float32(0)
    part_cnt = jnp.zeros((_K, 1), jnp.float32)
    for u in range(_U):
        idx_u, q_u, ploss_u, pcnt_u = _one(z_ref[u])
        idx_ref[u, 0, :] = idx_u
        out_ref[u] = q_u
        part_loss = part_loss + ploss_u
        part_cnt = part_cnt + pcnt_u

    @pl.when(b == 0)
    def _():
        loss_acc[0, 0] = part_loss
        cnt_acc[...] = part_cnt
    @pl.when(b > 0)
    def _():
        loss_acc[0, 0] += part_loss
        cnt_acc[...] += part_cnt

    @pl.when(b == _B // _U - 1)
    def _():
        loss_ref[0, 0] = (_COMMIT / (_N * _C)) * loss_acc[0, 0]
        avg = cnt_acc[...] / _N
        perp_ref[0, 0] = jnp.exp(-jnp.sum(avg * jnp.log(avg + 1e-10)))


def kernel(z, embedding):
    zv = z.reshape(_B, _C, _P)
    out_q, idx, loss, perp = pl.pallas_call(
        _vq_body,
        grid=(_B // _U,),
        in_specs=[
            pl.BlockSpec((_U, _C, _P), lambda b: (b, 0, 0)),
            pl.BlockSpec((_K, _C), lambda b: (0, 0)),
            pl.BlockSpec((_C, _K), lambda b: (0, 0)),
        ],
        out_specs=[
            pl.BlockSpec((_U, _C, _P), lambda b: (b, 0, 0)),
            pl.BlockSpec((_U, 1, _P), lambda b: (b, 0, 0)),
            pl.BlockSpec(memory_space=pltpu.SMEM),
            pl.BlockSpec(memory_space=pltpu.SMEM),
        ],
        out_shape=[
            jax.ShapeDtypeStruct((_B, _C, _P), jnp.float32),
            jax.ShapeDtypeStruct((_B, 1, _P), jnp.int32),
            jax.ShapeDtypeStruct((1, 1), jnp.float32),
            jax.ShapeDtypeStruct((1, 1), jnp.float32),
        ],
        scratch_shapes=[
            pltpu.SMEM((1, 1), jnp.float32),
            pltpu.VMEM((_K, 1), jnp.float32),
        ],
    )(zv, embedding, embedding.T)

    return (out_q.reshape(_B, _C, _H, _W),
            loss[0, 0],
            idx.reshape(_B, _H, _W),
            perp[0, 0])
